# baseline (speedup 1.0000x reference)
import functools

import jax
import jax.numpy as jnp
from jax import lax
from jax.experimental import pallas as pl
from jax.experimental.pallas import tpu as pltpu
from jax.experimental.pallas import tpu_sc as plsc

HIDDEN = 4096
N_TOK = 16384
NUM_CORES = 2
NUM_SUBCORES = 16
NW = NUM_CORES * NUM_SUBCORES
B_PER_W = N_TOK // NW
CHUNK = 4
N_CHUNKS = B_PER_W // CHUNK
NBUF = 4


def _build():
    mesh = plsc.VectorSubcoreMesh(core_axis_name="c", subcore_axis_name="s")

    @functools.partial(
        pl.kernel,
        mesh=mesh,
        out_type=jax.ShapeDtypeStruct((N_TOK, HIDDEN), jnp.float32),
        scratch_types=[
            pltpu.VMEM((N_CHUNKS, CHUNK), jnp.int32),
            pltpu.VMEM((NBUF, CHUNK, HIDDEN), jnp.float32),
        ] + [pltpu.SemaphoreType.DMA] * (2 * NBUF),
    )
    def emb(ids_hbm, table_hbm, out_hbm, idx_v, rows_v, *sems):
        gsem = list(sems[:NBUF])
        osem = list(sems[NBUF:])
        wid = lax.axis_index("s") * NUM_CORES + lax.axis_index("c")
        base = wid * B_PER_W
        pltpu.sync_copy(ids_hbm.at[pl.ds(wid * N_CHUNKS, N_CHUNKS)], idx_v)

        def g_desc(j, b):
            return pltpu.make_async_copy(
                table_hbm.at[idx_v.at[j]],
                rows_v.at[b],
                gsem[b],
            )

        def o_desc(j, b):
            return pltpu.make_async_copy(
                rows_v.at[b],
                out_hbm.at[pl.ds(base + j * CHUNK, CHUNK)],
                osem[b],
            )

        for b in range(NBUF):
            g_desc(b, b).start()

        def outer(i, carry):
            j0 = i * NBUF
            for b in range(NBUF):
                j = j0 + b
                g_desc(j, b).wait()
                o_desc(j, b).start()

                @pl.when(j + NBUF < N_CHUNKS)
                def _():
                    o_desc(j, b).wait()
                    g_desc(j + NBUF, b).start()

            return carry

        lax.fori_loop(0, N_CHUNKS // NBUF, outer, 0)

        for b in range(NBUF):
            o_desc(N_CHUNKS - NBUF + b, b).wait()

    return emb


_emb = _build()


def kernel(input_ids, embed_tokens_weight):
    b, s = input_ids.shape
    ids_flat = input_ids.reshape(N_TOK // CHUNK, CHUNK).astype(jnp.int32)
    out = _emb(ids_flat, embed_tokens_weight)
    return out.reshape(b, s, HIDDEN)


# speedup vs baseline: 1.0127x; 1.0127x over previous
import functools

import jax
import jax.numpy as jnp
from jax import lax
from jax.experimental import pallas as pl
from jax.experimental.pallas import tpu as pltpu
from jax.experimental.pallas import tpu_sc as plsc

HIDDEN = 4096
N_TOK = 16384
NUM_CORES = 2
NUM_SUBCORES = 16
NW = NUM_CORES * NUM_SUBCORES
B_PER_W = N_TOK // NW
CHUNK = 8
N_CHUNKS = B_PER_W // CHUNK
NBUF = 2


def _build():
    mesh = plsc.VectorSubcoreMesh(core_axis_name="c", subcore_axis_name="s")

    @functools.partial(
        pl.kernel,
        mesh=mesh,
        out_type=jax.ShapeDtypeStruct((N_TOK, HIDDEN), jnp.float32),
        scratch_types=[
            pltpu.VMEM((N_CHUNKS, CHUNK), jnp.int32),
            pltpu.VMEM((NBUF, CHUNK, HIDDEN), jnp.float32),
        ] + [pltpu.SemaphoreType.DMA] * (2 * NBUF),
    )
    def emb(ids_hbm, table_hbm, out_hbm, idx_v, rows_v, *sems):
        gsem = list(sems[:NBUF])
        osem = list(sems[NBUF:])
        wid = lax.axis_index("s") * NUM_CORES + lax.axis_index("c")
        base = wid * B_PER_W
        pltpu.sync_copy(ids_hbm.at[pl.ds(wid * N_CHUNKS, N_CHUNKS)], idx_v)

        def g_desc(j, b):
            return pltpu.make_async_copy(
                table_hbm.at[idx_v.at[j]],
                rows_v.at[b],
                gsem[b],
            )

        def o_desc(j, b):
            return pltpu.make_async_copy(
                rows_v.at[b],
                out_hbm.at[pl.ds(base + j * CHUNK, CHUNK)],
                osem[b],
            )

        for b in range(NBUF):
            g_desc(b, b).start()

        def outer(i, carry):
            j0 = i * NBUF
            for b in range(NBUF):
                j = j0 + b
                g_desc(j, b).wait()
                o_desc(j, b).start()

                @pl.when(j + NBUF < N_CHUNKS)
                def _():
                    o_desc(j, b).wait()
                    g_desc(j + NBUF, b).start()

            return carry

        lax.fori_loop(0, N_CHUNKS // NBUF, outer, 0)

        for b in range(NBUF):
            o_desc(N_CHUNKS - NBUF + b, b).wait()

    return emb


_emb = _build()


def kernel(input_ids, embed_tokens_weight):
    b, s = input_ids.shape
    ids_flat = input_ids.reshape(N_TOK // CHUNK, CHUNK).astype(jnp.int32)
    out = _emb(ids_flat, embed_tokens_weight)
    return out.reshape(b, s, HIDDEN)
